# baseline (speedup 1.0000x reference)
import jax
import jax.numpy as jnp
from jax.experimental import pallas as pl


def _body(h_ref, ta_ref, tb_ref, o_ref):
    i = pl.program_id(0)
    j = pl.program_id(1)
    hb = h_ref[...]
    m = jnp.max(hb, axis=1)
    ia = jax.lax.broadcasted_iota(jnp.int32, hb.shape, 1)
    aidx = jnp.min(jnp.where(hb == m[:, None], ia, 14), axis=1)
    ic = jax.lax.broadcasted_iota(jnp.int32, m.shape, 1)
    code = aidx * 16 + ic
    mx = jnp.max(m, axis=1, keepdims=True)
    k = jnp.min(jnp.where(m == mx, code, 4096), axis=1)
    a = k >> 4
    c = k & 15
    px = c.astype(jnp.float32) * 0.0625
    py = a.astype(jnp.float32) * 0.0625
    d0 = px - ta_ref[0]
    d1 = py - tb_ref[0]
    s = jnp.sum(d0 * d0 + d1 * d1)[None, None]

    @pl.when(jnp.logical_and(i == 0, j == 0))
    def _():
        o_ref[...] = jnp.zeros_like(o_ref)

    o_ref[...] += s


def kernel(o, h, t, v):
    B, Nj, col, _ = h.shape
    ht = jnp.transpose(h, (1, 2, 3, 0))
    tt = jnp.transpose(t, (1, 2, 0))
    bB = 8192 if B % 8192 == 0 else 128
    grid = (B // bB, Nj // 2)
    res = pl.pallas_call(
        _body,
        grid=grid,
        in_specs=[
            pl.BlockSpec((2, col, col, bB), lambda i, j: (j, 0, 0, i)),
            pl.BlockSpec((1, 2, bB), lambda i, j: (j, 0, i)),
            pl.BlockSpec((1, 2, bB), lambda i, j: (j + Nj // 2, 0, i)),
        ],
        out_specs=pl.BlockSpec((1, 1), lambda i, j: (0, 0)),
        out_shape=jax.ShapeDtypeStruct((1, 1), jnp.float32),
    )(ht, tt, tt)
    return res[0, 0] / jnp.float32(B * Nj)


# speedup vs baseline: 1.0937x; 1.0937x over previous
import jax
import jax.numpy as jnp
from jax.experimental import pallas as pl


def _body(h_ref, t_ref, o_ref):
    i = pl.program_id(0)
    hb = h_ref[...]
    m = jnp.max(hb, axis=1)
    ia = jax.lax.broadcasted_iota(jnp.int32, hb.shape, 1)
    aidx = jnp.min(jnp.where(hb == m[:, None], ia, 14), axis=1)
    ic = jax.lax.broadcasted_iota(jnp.int32, m.shape, 1)
    code = aidx * 16 + ic
    mx = jnp.max(m, axis=1, keepdims=True)
    k = jnp.min(jnp.where(m == mx, code, 4096), axis=1)
    a = k >> 4
    c = k & 15
    px = c.astype(jnp.float32) * 0.0625
    py = a.astype(jnp.float32) * 0.0625
    tt = t_ref[...].reshape(2 * t_ref.shape[0], t_ref.shape[2])
    d0 = px - tt[:14]
    d1 = py - tt[14:]
    s = jnp.sum(d0 * d0 + d1 * d1)[None, None]

    @pl.when(i == 0)
    def _():
        o_ref[...] = jnp.zeros_like(o_ref)

    o_ref[...] += s


def kernel(o, h, t, v):
    B, Nj, col, _ = h.shape
    ht = jnp.transpose(h, (1, 2, 3, 0))
    tt = jnp.transpose(t, (1, 2, 0))
    bB = 1024 if B % 1024 == 0 else 128
    grid = (B // bB,)
    res = pl.pallas_call(
        _body,
        grid=grid,
        in_specs=[
            pl.BlockSpec((Nj, col, col, bB), lambda i: (0, 0, 0, i)),
            pl.BlockSpec((Nj, 2, bB), lambda i: (0, 0, i)),
        ],
        out_specs=pl.BlockSpec((1, 1), lambda i: (0, 0)),
        out_shape=jax.ShapeDtypeStruct((1, 1), jnp.float32),
    )(ht, tt)
    return res[0, 0] / jnp.float32(B * Nj)
